# baseline (speedup 1.0000x reference)
import jax
import jax.numpy as jnp
from jax import lax
from jax.experimental import pallas as pl
from jax.experimental.pallas import tpu as pltpu


def _fused_kernel(idx_ref,
                  emb_ref,
                  w_ih_ref, w_hh_ref, b_ref,
                  w1_ref, b1_ref,
                  w2_ref, b2_ref,
                  w3_ref, b3_ref,
                  out_ref,
                  x_buf, dma_sem):
    M = idx_ref.shape[0]
    E = w_ih_ref.shape[0]
    H = w_hh_ref.shape[0]
    B = out_ref.shape[0]
    B2 = 2 * B
    S = M // B2
    H4 = 4 * H
    Eh = E // 2

    for mi in range(M):
        pltpu.make_async_copy(emb_ref.at[idx_ref[mi]], x_buf.at[mi],
                              dma_sem).start()

    gate_q = lax.broadcasted_iota(jnp.int32, (1, H4), 1) // H
    gscale = jnp.where(gate_q == 2, 1.0, 0.5).astype(jnp.float32)
    w_hh_s = w_hh_ref[...] * gscale
    b_s = b_ref[...] * gscale
    w_ih_lo = w_ih_ref[:Eh, :] * gscale
    w_ih_hi = w_ih_ref[Eh:, :] * gscale

    pltpu.make_async_copy(emb_ref.at[pl.ds(0, M)], x_buf.at[pl.ds(0, M)],
                          dma_sem).wait()

    x_lo = x_buf[:, 0, :]
    x_hi = x_buf[:, 1, :]
    gx = (jnp.dot(x_lo, w_ih_lo, preferred_element_type=jnp.float32)
          + jnp.dot(x_hi, w_ih_hi, preferred_element_type=jnp.float32)
          + b_s)

    def gates(z):
        th = jnp.tanh(z)
        return (th[:, :H], th[:, H:2 * H], th[:, 2 * H:3 * H], th[:, 3 * H:])

    i_g, _, g_g, o_g = gates(gx[(S - 1) * B2:S * B2, :])
    c = (0.5 * i_g + 0.5) * g_g
    h = (0.5 * o_g + 0.5) * jnp.tanh(c)
    for t in range(S - 2, -1, -1):
        z = gx[t * B2:(t + 1) * B2, :] + jnp.dot(
            h, w_hh_s, preferred_element_type=jnp.float32)
        i_g, f_g, g_g, o_g = gates(z)
        c = (0.5 * f_g + 0.5) * c + (0.5 * i_g + 0.5) * g_g
        h = (0.5 * o_g + 0.5) * jnp.tanh(c)

    y = jnp.maximum(
        jnp.dot(h[:B, :], w1_ref[:H, :], preferred_element_type=jnp.float32)
        + jnp.dot(h[B:, :], w1_ref[H:, :], preferred_element_type=jnp.float32)
        + b1_ref[...], 0.0)
    y = jnp.maximum(
        jnp.dot(y, w2_ref[...], preferred_element_type=jnp.float32)
        + b2_ref[...], 0.0)
    y = jnp.maximum(
        jnp.dot(y, w3_ref[...], preferred_element_type=jnp.float32)
        + b3_ref[...], 0.0)
    out_ref[...] = y.astype(out_ref.dtype)


@jax.jit
def _forward(embedding, w_ih_rev, w_hh_rev, b_lstm_rev,
             w1, b1, w2, b2, w3, b3, premise, hypothesis):
    B, S = premise.shape
    V, E = embedding.shape
    C = w3.shape[1]
    M = S * 2 * B

    idx = jnp.concatenate([premise, hypothesis], axis=0).astype(jnp.int32)
    idx_flat = jnp.transpose(idx, (1, 0)).reshape(M)

    emb3 = embedding.reshape(V, E // 128, 128)

    dense = (w_ih_rev, w_hh_rev, b_lstm_rev, w1, b1, w2, b2, w3, b3)

    def vmem_spec(a):
        nd = a.ndim
        return pl.BlockSpec(a.shape, lambda i, nd=nd: (0,) * nd)

    return pl.pallas_call(
        _fused_kernel,
        out_shape=jax.ShapeDtypeStruct((B, C), jnp.float32),
        grid=(1,),
        in_specs=[pl.BlockSpec(memory_space=pltpu.MemorySpace.SMEM),
                  pl.BlockSpec(memory_space=pltpu.MemorySpace.HBM)]
                 + [vmem_spec(a) for a in dense],
        out_specs=pl.BlockSpec((B, C), lambda i: (0, 0)),
        scratch_shapes=[pltpu.VMEM((M, E // 128, 128), jnp.float32),
                        pltpu.SemaphoreType.DMA],
        compiler_params=pltpu.CompilerParams(
            dimension_semantics=("arbitrary",)),
    )(idx_flat, emb3, *dense)


def kernel(embedding, w_ih_rev, w_hh_rev, b_lstm_rev,
           w1, b1, w2, b2, w3, b3, premise, hypothesis):
    return _forward(embedding, w_ih_rev, w_hh_rev, b_lstm_rev,
                    w1, b1, w2, b2, w3, b3, premise, hypothesis)


# speedup vs baseline: 2.7529x; 2.7529x over previous
import jax
import jax.numpy as jnp
from jax import lax
from jax.experimental import pallas as pl
from jax.experimental.pallas import tpu as pltpu


def _fused_kernel(idx_ref,
                  emb_ref,
                  w_ih_ref, w_hh_ref, b_ref,
                  w1_ref, b1_ref,
                  w2_ref, b2_ref,
                  w3_ref, b3_ref,
                  out_ref,
                  x_buf, dma_sem):
    M = idx_ref.shape[0]
    E = w_ih_ref.shape[0]
    H = w_hh_ref.shape[0]
    B = out_ref.shape[0]
    B2 = 2 * B
    S = M // B2
    H4 = 4 * H

    for mi in range(M):
        pltpu.make_async_copy(emb_ref.at[pl.ds(idx_ref[mi], 1), :],
                              x_buf.at[mi], dma_sem).start()

    gate_q = lax.broadcasted_iota(jnp.int32, (1, H4), 1) // H
    gscale = jnp.where(gate_q == 2, 1.0, 0.5).astype(jnp.float32)
    w_hh_s = w_hh_ref[...] * gscale
    b_s = b_ref[...] * gscale
    w_ih_s = w_ih_ref[...] * gscale

    pltpu.make_async_copy(emb_ref.at[pl.ds(0, M), :],
                          x_buf.at[pl.ds(0, M), 0], dma_sem).wait()

    x = x_buf[:, 0, :]
    gx = (jnp.dot(x, w_ih_s, preferred_element_type=jnp.float32)
          + b_s)

    def gates(z):
        th = jnp.tanh(z)
        return (th[:, :H], th[:, H:2 * H], th[:, 2 * H:3 * H], th[:, 3 * H:])

    i_g, _, g_g, o_g = gates(gx[(S - 1) * B2:S * B2, :])
    c = (0.5 * i_g + 0.5) * g_g
    h = (0.5 * o_g + 0.5) * jnp.tanh(c)
    for t in range(S - 2, -1, -1):
        z = gx[t * B2:(t + 1) * B2, :] + jnp.dot(
            h, w_hh_s, preferred_element_type=jnp.float32)
        i_g, f_g, g_g, o_g = gates(z)
        c = (0.5 * f_g + 0.5) * c + (0.5 * i_g + 0.5) * g_g
        h = (0.5 * o_g + 0.5) * jnp.tanh(c)

    y = jnp.maximum(
        jnp.dot(h[:B, :], w1_ref[:H, :], preferred_element_type=jnp.float32)
        + jnp.dot(h[B:, :], w1_ref[H:, :], preferred_element_type=jnp.float32)
        + b1_ref[...], 0.0)
    y = jnp.maximum(
        jnp.dot(y, w2_ref[...], preferred_element_type=jnp.float32)
        + b2_ref[...], 0.0)
    y = jnp.maximum(
        jnp.dot(y, w3_ref[...], preferred_element_type=jnp.float32)
        + b3_ref[...], 0.0)
    out_ref[...] = y.astype(out_ref.dtype)


@jax.jit
def _forward(embedding, w_ih_rev, w_hh_rev, b_lstm_rev,
             w1, b1, w2, b2, w3, b3, premise, hypothesis):
    B, S = premise.shape
    V, E = embedding.shape
    C = w3.shape[1]
    M = S * 2 * B

    idx = jnp.concatenate([premise, hypothesis], axis=0).astype(jnp.int32)
    idx_flat = jnp.transpose(idx, (1, 0)).reshape(M)

    dense = (w_ih_rev, w_hh_rev, b_lstm_rev, w1, b1, w2, b2, w3, b3)

    def vmem_spec(a):
        nd = a.ndim
        return pl.BlockSpec(a.shape, lambda i, nd=nd: (0,) * nd)

    return pl.pallas_call(
        _fused_kernel,
        out_shape=jax.ShapeDtypeStruct((B, C), jnp.float32),
        grid=(1,),
        in_specs=[pl.BlockSpec(memory_space=pltpu.MemorySpace.SMEM),
                  pl.BlockSpec(memory_space=pltpu.MemorySpace.HBM)]
                 + [vmem_spec(a) for a in dense],
        out_specs=pl.BlockSpec((B, C), lambda i: (0, 0)),
        scratch_shapes=[pltpu.VMEM((M, 1, E), jnp.float32),
                        pltpu.SemaphoreType.DMA],
        compiler_params=pltpu.CompilerParams(
            dimension_semantics=("arbitrary",)),
    )(idx_flat, embedding, *dense)


def kernel(embedding, w_ih_rev, w_hh_rev, b_lstm_rev,
           w1, b1, w2, b2, w3, b3, premise, hypothesis):
    return _forward(embedding, w_ih_rev, w_hh_rev, b_lstm_rev,
                    w1, b1, w2, b2, w3, b3, premise, hypothesis)


# speedup vs baseline: 3.1547x; 1.1459x over previous
import jax
import jax.numpy as jnp
from jax import lax
from jax.experimental import pallas as pl
from jax.experimental.pallas import tpu as pltpu


def _fused_kernel(prem_ref, hyp_ref,
                  emb_ref,
                  w_ih_ref, w_hh_ref, b_ref,
                  w1_ref, b1_ref,
                  w2_ref, b2_ref,
                  w3_ref, b3_ref,
                  out_ref,
                  x_buf, dma_sem):
    B, S = prem_ref.shape
    E = w_ih_ref.shape[0]
    H = w_hh_ref.shape[0]
    B2 = 2 * B
    M = S * B2
    H4 = 4 * H

    for t in range(S):
        for r in range(B2):
            tok = prem_ref[r, t] if r < B else hyp_ref[r - B, t]
            pltpu.make_async_copy(emb_ref.at[pl.ds(tok, 1), :],
                                  x_buf.at[t * B2 + r], dma_sem).start()

    gate_q = lax.broadcasted_iota(jnp.int32, (1, H4), 1) // H
    gscale = jnp.where(gate_q == 2, 1.0, 0.5).astype(jnp.float32)
    w_hh_s = w_hh_ref[...] * gscale
    b_s = b_ref[...] * gscale
    w_ih_s = w_ih_ref[...] * gscale

    pltpu.make_async_copy(emb_ref.at[pl.ds(0, M), :],
                          x_buf.at[pl.ds(0, M), 0], dma_sem).wait()

    x = x_buf[:, 0, :]
    gx = (jnp.dot(x, w_ih_s, preferred_element_type=jnp.float32)
          + b_s)

    def gates(z):
        th = jnp.tanh(z)
        return (th[:, :H], th[:, H:2 * H], th[:, 2 * H:3 * H], th[:, 3 * H:])

    i_g, _, g_g, o_g = gates(gx[(S - 1) * B2:S * B2, :])
    c = (0.5 * i_g + 0.5) * g_g
    h = (0.5 * o_g + 0.5) * jnp.tanh(c)
    for t in range(S - 2, -1, -1):
        z = gx[t * B2:(t + 1) * B2, :] + jnp.dot(
            h, w_hh_s, preferred_element_type=jnp.float32)
        i_g, f_g, g_g, o_g = gates(z)
        c = (0.5 * f_g + 0.5) * c + (0.5 * i_g + 0.5) * g_g
        h = (0.5 * o_g + 0.5) * jnp.tanh(c)

    y = jnp.maximum(
        jnp.dot(h[:B, :], w1_ref[:H, :], preferred_element_type=jnp.float32)
        + jnp.dot(h[B:, :], w1_ref[H:, :], preferred_element_type=jnp.float32)
        + b1_ref[...], 0.0)
    y = jnp.maximum(
        jnp.dot(y, w2_ref[...], preferred_element_type=jnp.float32)
        + b2_ref[...], 0.0)
    y = jnp.maximum(
        jnp.dot(y, w3_ref[...], preferred_element_type=jnp.float32)
        + b3_ref[...], 0.0)
    out_ref[...] = y.astype(out_ref.dtype)


@jax.jit
def _forward(embedding, w_ih_rev, w_hh_rev, b_lstm_rev,
             w1, b1, w2, b2, w3, b3, premise, hypothesis):
    B, S = premise.shape
    V, E = embedding.shape
    C = w3.shape[1]
    M = S * 2 * B

    dense = (w_ih_rev, w_hh_rev, b_lstm_rev, w1, b1, w2, b2, w3, b3)

    def vmem_spec(a):
        nd = a.ndim
        return pl.BlockSpec(a.shape, lambda i, nd=nd: (0,) * nd)

    smem = pl.BlockSpec(memory_space=pltpu.MemorySpace.SMEM)
    return pl.pallas_call(
        _fused_kernel,
        out_shape=jax.ShapeDtypeStruct((B, C), jnp.float32),
        grid=(1,),
        in_specs=[smem, smem,
                  pl.BlockSpec(memory_space=pltpu.MemorySpace.HBM)]
                 + [vmem_spec(a) for a in dense],
        out_specs=pl.BlockSpec((B, C), lambda i: (0, 0)),
        scratch_shapes=[pltpu.VMEM((M, 1, E), jnp.float32),
                        pltpu.SemaphoreType.DMA],
        compiler_params=pltpu.CompilerParams(
            dimension_semantics=("arbitrary",)),
    )(premise, hypothesis, embedding, *dense)


def kernel(embedding, w_ih_rev, w_hh_rev, b_lstm_rev,
           w1, b1, w2, b2, w3, b3, premise, hypothesis):
    return _forward(embedding, w_ih_rev, w_hh_rev, b_lstm_rev,
                    w1, b1, w2, b2, w3, b3, premise, hypothesis)
